# baseline (speedup 1.0000x reference)
import functools

import jax
import jax.numpy as jnp
from jax import lax
from jax.experimental import pallas as pl
from jax.experimental.pallas import tpu as pltpu
from jax.experimental.pallas import tpu_sc as plsc

_DIM = 64
_SCALE = 8.0
_LANES = 16
_NC = 2
_NS = 16
_NW = _NC * _NS
_CHUNK = 128


def _sc_embed(idx_t, table):
    seq, batch = idx_t.shape
    kpl = batch // _CHUNK
    nchunks = seq * kpl
    cpw = nchunks // _NW
    mesh = plsc.VectorSubcoreMesh(core_axis_name="c", subcore_axis_name="s")

    @functools.partial(
        pl.kernel,
        mesh=mesh,
        out_type=jax.ShapeDtypeStruct((seq, _DIM, batch), jnp.float32),
        compiler_params=pltpu.CompilerParams(
            use_tc_tiling_on_sc=False, needs_layout_passes=False
        ),
        scratch_types=[
            pltpu.VMEM((_CHUNK,), jnp.int32),
            pltpu.VMEM((_CHUNK,), jnp.int32),
            pltpu.VMEM((_CHUNK,), jnp.int32),
            pltpu.VMEM((_CHUNK,), jnp.int32),
            pltpu.VMEM((_CHUNK, _DIM), jnp.float32),
            pltpu.VMEM((_CHUNK, _DIM), jnp.float32),
            pltpu.VMEM((_DIM, _CHUNK), jnp.float32),
            pltpu.VMEM((_DIM, _CHUNK), jnp.float32),
            pltpu.SemaphoreType.DMA,
            pltpu.SemaphoreType.DMA,
            pltpu.SemaphoreType.DMA,
            pltpu.SemaphoreType.DMA,
            pltpu.SemaphoreType.DMA,
            pltpu.SemaphoreType.DMA,
            pltpu.SemaphoreType.DMA,
            pltpu.SemaphoreType.DMA,
        ],
    )
    def body(idx_hbm, table_hbm, out_hbm, i0, i1, i2, i3, g0, g1, t0, t1,
             is0, is1, is2, is3, gs0, gs1, ss0, ss1):
        wid = lax.axis_index("s") * _NC + lax.axis_index("c")
        cbase = wid * cpw

        ibufs = (i0, i1, i2, i3)
        isems = (is0, is1, is2, is3)
        gbufs = (g0, g1)
        gsems = (gs0, gs1)
        tbufs = (t0, t1)
        ssems = (ss0, ss1)

        def lk(j):
            c = cbase + j
            return c // kpl, lax.rem(c, kpl)

        def idx_start(j, islot):
            l, k = lk(j)
            pltpu.async_copy(idx_hbm.at[l, pl.ds(k * _CHUNK, _CHUNK)],
                             ibufs[islot], isems[islot])

        def idx_wait(islot):
            pltpu.make_async_copy(idx_hbm.at[0, pl.ds(0, _CHUNK)],
                                  ibufs[islot], isems[islot]).wait()

        def gather_start(islot, gslot):
            pltpu.async_copy(table_hbm.at[ibufs[islot]], gbufs[gslot],
                             gsems[gslot])

        def gather_wait(gslot):
            pltpu.make_async_copy(table_hbm.at[ibufs[0]], gbufs[gslot],
                                  gsems[gslot]).wait()

        def store_start(j, gslot):
            l, k = lk(j)
            pltpu.async_copy(tbufs[gslot],
                             out_hbm.at[l, pl.ds(0, _DIM),
                                        pl.ds(k * _CHUNK, _CHUNK)],
                             ssems[gslot])

        def store_wait(gslot):
            pltpu.make_async_copy(tbufs[gslot],
                                  out_hbm.at[0, pl.ds(0, _DIM),
                                             pl.ds(0, _CHUNK)],
                                  ssems[gslot]).wait()

        def transpose_scale(gslot):
            g = gbufs[gslot]
            t = tbufs[gslot]
            iot = lax.iota(jnp.int32, _LANES)

            def f_body(f, carry):
                cols = jnp.full((_LANES,), 0, jnp.int32) + f
                for c in range(_CHUNK // _LANES):
                    rows = iot + (c * _LANES)
                    v = plsc.load_gather(g, [rows, cols])
                    t[f, pl.ds(c * _LANES, _LANES)] = v * _SCALE
                return carry

            lax.fori_loop(0, _DIM, f_body, 0)

        for j in range(4):
            idx_start(j, j)
        idx_wait(0)
        gather_start(0, 0)
        idx_wait(1)
        gather_start(1, 1)

        def step(tt, carry):
            for u in range(4):
                j = 4 * tt + u
                gslot = u % 2

                @pl.when(j >= 2)
                def _():
                    store_wait(gslot)

                gather_wait(gslot)

                @pl.when(j + 4 < cpw)
                def _():
                    idx_start(j + 4, u)

                transpose_scale(gslot)
                store_start(j, gslot)

                @pl.when(j + 2 < cpw)
                def _():
                    idx_wait((u + 2) % 4)
                    gather_start((u + 2) % 4, gslot)
            return carry

        lax.fori_loop(0, cpw // 4, step, 0)
        store_wait(0)
        store_wait(1)

    return body(idx_t, table)


def kernel(x, table):
    b, l = x.shape
    idx_t = x.T.astype(jnp.int32)
    out_phys = _sc_embed(idx_t, table)
    return jnp.transpose(out_phys, (2, 0, 1))


# speedup vs baseline: 1.6693x; 1.6693x over previous
import functools

import jax
import jax.numpy as jnp
from jax import lax
from jax.experimental import pallas as pl
from jax.experimental.pallas import tpu as pltpu
from jax.experimental.pallas import tpu_sc as plsc

_DIM = 64
_SCALE = 8.0
_LANES = 16
_NC = 2
_NS = 16
_NW = _NC * _NS
_CHUNK = 128


def _sc_embed(idx_t, table):
    seq, batch = idx_t.shape
    kpl = batch // _CHUNK
    nchunks = seq * kpl
    cpw = nchunks // _NW
    mesh = plsc.VectorSubcoreMesh(core_axis_name="c", subcore_axis_name="s")

    @functools.partial(
        pl.kernel,
        mesh=mesh,
        out_type=jax.ShapeDtypeStruct(
            (seq, _DIM // 8, batch // _CHUNK, 8, _CHUNK), jnp.float32
        ),
        compiler_params=pltpu.CompilerParams(
            use_tc_tiling_on_sc=False, needs_layout_passes=False
        ),
        scratch_types=[
            pltpu.VMEM((_CHUNK,), jnp.int32),
            pltpu.VMEM((_CHUNK,), jnp.int32),
            pltpu.VMEM((_CHUNK,), jnp.int32),
            pltpu.VMEM((_CHUNK,), jnp.int32),
            pltpu.VMEM((_CHUNK, _DIM), jnp.float32),
            pltpu.VMEM((_CHUNK, _DIM), jnp.float32),
            pltpu.VMEM((_DIM // 8, 8, _CHUNK), jnp.float32),
            pltpu.VMEM((_DIM // 8, 8, _CHUNK), jnp.float32),
            pltpu.SemaphoreType.DMA,
            pltpu.SemaphoreType.DMA,
            pltpu.SemaphoreType.DMA,
            pltpu.SemaphoreType.DMA,
            pltpu.SemaphoreType.DMA,
            pltpu.SemaphoreType.DMA,
            pltpu.SemaphoreType.DMA,
            pltpu.SemaphoreType.DMA,
        ],
    )
    def body(idx_hbm, table_hbm, out_hbm, i0, i1, i2, i3, g0, g1, t0, t1,
             is0, is1, is2, is3, gs0, gs1, ss0, ss1):
        wid = lax.axis_index("s") * _NC + lax.axis_index("c")
        cbase = wid * cpw

        ibufs = (i0, i1, i2, i3)
        isems = (is0, is1, is2, is3)
        gbufs = (g0, g1)
        gsems = (gs0, gs1)
        tbufs = (t0, t1)
        ssems = (ss0, ss1)

        def lk(j):
            c = cbase + j
            return c // kpl, lax.rem(c, kpl)

        def idx_start(j, islot):
            l, k = lk(j)
            pltpu.async_copy(idx_hbm.at[l, pl.ds(k * _CHUNK, _CHUNK)],
                             ibufs[islot], isems[islot])

        def idx_wait(islot):
            pltpu.make_async_copy(idx_hbm.at[0, pl.ds(0, _CHUNK)],
                                  ibufs[islot], isems[islot]).wait()

        def gather_start(islot, gslot):
            pltpu.async_copy(table_hbm.at[ibufs[islot]], gbufs[gslot],
                             gsems[gslot])

        def gather_wait(gslot):
            pltpu.make_async_copy(table_hbm.at[ibufs[0]], gbufs[gslot],
                                  gsems[gslot]).wait()

        def store_start(j, gslot):
            l, k = lk(j)
            pltpu.async_copy(tbufs[gslot], out_hbm.at[l, :, k],
                             ssems[gslot])

        def store_wait(gslot):
            pltpu.make_async_copy(tbufs[gslot], out_hbm.at[0, :, 0],
                                  ssems[gslot]).wait()

        def transpose_scale(gslot):
            g = gbufs[gslot]
            t = tbufs[gslot]
            iot = lax.iota(jnp.int32, _LANES)

            @plsc.parallel_loop(0, _DIM, unroll=8)
            def _(f):
                tr = lax.div(f, 8)
                fr = lax.rem(f, 8)
                cols = iot * 0 + f
                for c in range(_CHUNK // _LANES):
                    rows = iot + (c * _LANES)
                    v = plsc.load_gather(g, [rows, cols])
                    t[tr, fr, pl.ds(c * _LANES, _LANES)] = v * _SCALE

        for j in range(4):
            idx_start(j, j)
        idx_wait(0)
        gather_start(0, 0)
        idx_wait(1)
        gather_start(1, 1)

        def step(tt, carry):
            for u in range(4):
                j = 4 * tt + u
                gslot = u % 2

                @pl.when(j >= 2)
                def _():
                    store_wait(gslot)

                gather_wait(gslot)

                @pl.when(j + 4 < cpw)
                def _():
                    idx_start(j + 4, u)

                transpose_scale(gslot)
                store_start(j, gslot)

                @pl.when(j + 2 < cpw)
                def _():
                    idx_wait((u + 2) % 4)
                    gather_start((u + 2) % 4, gslot)
            return carry

        lax.fori_loop(0, cpw // 4, step, 0)
        store_wait(0)
        store_wait(1)

    return body(idx_t, table)


def kernel(x, table):
    b, l = x.shape
    idx_t = x.T.astype(jnp.int32)
    out5 = _sc_embed(idx_t, table)
    outp = jnp.transpose(out5, (2, 4, 0, 1, 3))
    return outp.reshape(b, l, _DIM)


# speedup vs baseline: 2.7088x; 1.6227x over previous
import functools

import jax
import jax.numpy as jnp
from jax import lax
from jax.experimental import pallas as pl
from jax.experimental.pallas import tpu as pltpu
from jax.experimental.pallas import tpu_sc as plsc

_DIM = 64
_SCALE = 8.0
_LANES = 16
_NC = 2
_NS = 16
_NW = _NC * _NS
_CHUNK = 128


def _sc_embed(idx_t, table):
    seq, batch = idx_t.shape
    kpl = batch // _CHUNK
    nchunks = seq * kpl
    cpw = nchunks // _NW
    mesh = plsc.VectorSubcoreMesh(core_axis_name="c", subcore_axis_name="s")

    @functools.partial(
        pl.kernel,
        mesh=mesh,
        out_type=jax.ShapeDtypeStruct(
            (seq, _DIM // 8, batch // _CHUNK, 8, _CHUNK), jnp.float32
        ),
        compiler_params=pltpu.CompilerParams(
            use_tc_tiling_on_sc=False, needs_layout_passes=False
        ),
        scratch_types=[
            pltpu.VMEM((_CHUNK,), jnp.int32),
            pltpu.VMEM((_CHUNK,), jnp.int32),
            pltpu.VMEM((_CHUNK,), jnp.int32),
            pltpu.VMEM((_CHUNK,), jnp.int32),
            pltpu.VMEM((_CHUNK, _DIM), jnp.float32),
            pltpu.VMEM((_CHUNK, _DIM), jnp.float32),
            pltpu.VMEM((_DIM // 8, 8, _CHUNK + 1), jnp.float32),
            pltpu.VMEM((_DIM // 8, 8, _CHUNK + 1), jnp.float32),
            pltpu.SemaphoreType.DMA,
            pltpu.SemaphoreType.DMA,
            pltpu.SemaphoreType.DMA,
            pltpu.SemaphoreType.DMA,
            pltpu.SemaphoreType.DMA,
            pltpu.SemaphoreType.DMA,
            pltpu.SemaphoreType.DMA,
            pltpu.SemaphoreType.DMA,
        ],
    )
    def body(idx_hbm, table_hbm, out_hbm, i0, i1, i2, i3, g0, g1, t0, t1,
             is0, is1, is2, is3, gs0, gs1, ss0, ss1):
        wid = lax.axis_index("s") * _NC + lax.axis_index("c")
        cbase = wid * cpw

        ibufs = (i0, i1, i2, i3)
        isems = (is0, is1, is2, is3)
        gbufs = (g0, g1)
        gsems = (gs0, gs1)
        tbufs = (t0, t1)
        ssems = (ss0, ss1)

        def lk(j):
            c = cbase + j
            return c // kpl, lax.rem(c, kpl)

        def idx_start(j, islot):
            l, k = lk(j)
            pltpu.async_copy(idx_hbm.at[l, pl.ds(k * _CHUNK, _CHUNK)],
                             ibufs[islot], isems[islot])

        def idx_wait(islot):
            pltpu.make_async_copy(idx_hbm.at[0, pl.ds(0, _CHUNK)],
                                  ibufs[islot], isems[islot]).wait()

        def gather_start(islot, gslot):
            pltpu.async_copy(table_hbm.at[ibufs[islot]], gbufs[gslot],
                             gsems[gslot])

        def gather_wait(gslot):
            pltpu.make_async_copy(table_hbm.at[ibufs[0]], gbufs[gslot],
                                  gsems[gslot]).wait()

        def store_start(j, gslot):
            l, k = lk(j)
            pltpu.async_copy(tbufs[gslot].at[:, :, pl.ds(0, _CHUNK)],
                             out_hbm.at[l, :, k], ssems[gslot])

        def store_wait(gslot):
            pltpu.make_async_copy(tbufs[gslot].at[:, :, pl.ds(0, _CHUNK)],
                                  out_hbm.at[0, :, 0],
                                  ssems[gslot]).wait()

        def transpose_scale(gslot):
            g = gbufs[gslot]
            t = tbufs[gslot]
            iot = lax.iota(jnp.int32, _LANES)

            @plsc.parallel_loop(0, _CHUNK, unroll=8)
            def _(k):
                ks = iot * 0 + k
                for c in range(_DIM // _LANES):
                    f16 = iot + (c * _LANES)
                    trs = lax.shift_right_logical(f16, 3)
                    frs = f16 & 7
                    v = g[k, pl.ds(c * _LANES, _LANES)] * _SCALE
                    plsc.store_scatter(t, [trs, frs, ks], v)

        for j in range(4):
            idx_start(j, j)
        idx_wait(0)
        gather_start(0, 0)
        idx_wait(1)
        gather_start(1, 1)

        def step(tt, carry):
            for u in range(4):
                j = 4 * tt + u
                gslot = u % 2

                @pl.when(j >= 2)
                def _():
                    store_wait(gslot)

                gather_wait(gslot)

                @pl.when(j + 4 < cpw)
                def _():
                    idx_start(j + 4, u)

                transpose_scale(gslot)
                store_start(j, gslot)

                @pl.when(j + 2 < cpw)
                def _():
                    idx_wait((u + 2) % 4)
                    gather_start((u + 2) % 4, gslot)
            return carry

        lax.fori_loop(0, cpw // 4, step, 0)
        store_wait(0)
        store_wait(1)

    return body(idx_t, table)


def kernel(x, table):
    b, l = x.shape
    idx_t = x.T.astype(jnp.int32)
    out5 = _sc_embed(idx_t, table)
    outp = jnp.transpose(out5, (2, 4, 0, 1, 3))
    return outp.reshape(b, l, _DIM)
